# baseline (speedup 1.0000x reference)
import functools

import jax
import jax.numpy as jnp
from jax import lax
from jax.experimental import pallas as pl
from jax.experimental.pallas import tpu as pltpu
from jax.experimental.pallas import tpu_sc as plsc

L = 16
CH = 128
CH2 = 80
BR = 1000


def _prep_body(f_ref, wdt_ref, w2t_ref, b_ref, a_ref, bt_ref):
    f = f_ref[...]
    a_ref[...] = (
        jnp.dot(f, wdt_ref[...], preferred_element_type=jnp.float32) + b_ref[...]
    )
    bt_ref[...] = jnp.dot(f, w2t_ref[...], preferred_element_type=jnp.float32)


def _prep(feature, wdt, w2t, b2d):
    n, d = feature.shape
    grid = (n // BR,)
    return pl.pallas_call(
        _prep_body,
        grid=grid,
        in_specs=[
            pl.BlockSpec((BR, d), lambda i: (i, 0)),
            pl.BlockSpec((d, d), lambda i: (0, 0)),
            pl.BlockSpec((d, d), lambda i: (0, 0)),
            pl.BlockSpec((1, d), lambda i: (0, 0)),
        ],
        out_specs=[
            pl.BlockSpec((BR, d), lambda i: (i, 0)),
            pl.BlockSpec((BR, d), lambda i: (i, 0)),
        ],
        out_shape=[jax.ShapeDtypeStruct((n, d), jnp.float32)] * 2,
    )(feature, wdt, w2t, b2d)


def _stats(a_tab, b_tab, cntd, cnts, cross, gamma2d, beta2d, num_edges):
    n, d = a_tab.shape

    def body(a_ref, b_ref, cd_ref, cs_ref, cr_ref, g_ref, be_ref, st_ref):
        a = a_ref[...]
        bb = b_ref[...]
        cd = cd_ref[...]
        cs = cs_ref[...]
        dot_n = lambda u, v: lax.dot_general(
            u, v, (((0,), (0,)), ((), ())), preferred_element_type=jnp.float32
        ).reshape(1, d)
        sum_a = dot_n(cd, a)
        sum_b = dot_n(cs, bb)
        sum_a2 = dot_n(cd, a * a)
        sum_b2 = dot_n(cs, bb * bb)
        cross = jnp.sum(cr_ref[...], axis=0, keepdims=True)
        mean = (sum_a + sum_b) / num_edges
        ex2 = (sum_a2 + 2.0 * cross + sum_b2) / num_edges
        var = ex2 - mean * mean
        s = g_ref[...] * lax.rsqrt(var + 1e-5)
        t = be_ref[...] - mean * s
        st_ref[...] = jnp.concatenate([s, t], axis=0)

    return pl.pallas_call(
        body,
        out_shape=jax.ShapeDtypeStruct((2, d), jnp.float32),
    )(a_tab, b_tab, cntd, cnts, cross, gamma2d, beta2d)


def _count(dst, n):
    e = dst.shape[0]
    eb = 2000
    grid = (e // eb,)
    dst3 = dst.reshape(e // eb, 1, eb)

    def body(d_ref, o_ref):
        i = pl.program_id(0)
        idx = d_ref[0, 0, :]
        hi = (idx >> 7).astype(jnp.int32)
        lo = (idx & 127).astype(jnp.int32)
        cols = lax.broadcasted_iota(jnp.int32, (eb, 128), 1)
        oh_hi = (hi[:, None] == cols).astype(jnp.float32)
        oh_lo = (lo[:, None] == cols).astype(jnp.float32)
        blk = lax.dot_general(
            oh_hi, oh_lo, (((0,), (0,)), ((), ())),
            preferred_element_type=jnp.float32,
        )

        @pl.when(i == 0)
        def _():
            o_ref[...] = jnp.zeros_like(o_ref)

        o_ref[...] += blk

    cnt2d = pl.pallas_call(
        body,
        grid=grid,
        in_specs=[pl.BlockSpec((1, 1, eb), lambda i: (i, 0, 0))],
        out_specs=pl.BlockSpec((128, 128), lambda i: (0, 0)),
        out_shape=jax.ShapeDtypeStruct((128, 128), jnp.float32),
    )(dst3)
    return cnt2d.reshape(-1)[:n].reshape(n, 1)


def _finalize(p0, p1, cnt2):
    n, d = p0.shape
    grid = (n // BR,)

    def body(p0_ref, p1_ref, c_ref, o_ref):
        p = p0_ref[...] + p1_ref[...]
        cnt = jnp.maximum(c_ref[...], 1.0)
        o_ref[...] = p * (1.0 / cnt)

    return pl.pallas_call(
        body,
        grid=grid,
        in_specs=[
            pl.BlockSpec((BR, d), lambda i: (i, 0)),
            pl.BlockSpec((BR, d), lambda i: (i, 0)),
            pl.BlockSpec((BR, 1), lambda i: (i, 0)),
        ],
        out_specs=pl.BlockSpec((BR, d), lambda i: (i, 0)),
        out_shape=jax.ShapeDtypeStruct((n, d), jnp.float32),
    )(p0, p1, cnt2)


def _pass1(a_tab, b_tab, src, dst):
    n, d = a_tab.shape
    e = src.shape[0]
    dsub = d // L
    info = plsc.get_sparse_core_info()
    nc, ns = info.num_cores, info.num_subcores
    nw = nc * ns
    epw = e // nw
    nfull = epw // CH
    tail = epw % CH
    npair = nfull // 2
    odd = nfull % 2
    mesh = plsc.VectorSubcoreMesh(core_axis_name="c", subcore_axis_name="s")

    slot_types = [
        pltpu.VMEM((CH,), jnp.int32),
        pltpu.VMEM((CH,), jnp.int32),
        pltpu.VMEM((CH, d), jnp.float32),
        pltpu.VMEM((CH, d), jnp.float32),
        pltpu.SemaphoreType.DMA,
        pltpu.SemaphoreType.DMA,
        pltpu.SemaphoreType.DMA,
        pltpu.SemaphoreType.DMA,
    ]
    tw = tail if tail else 8
    tail_types = [
        pltpu.VMEM((tw,), jnp.int32),
        pltpu.VMEM((tw,), jnp.int32),
        pltpu.VMEM((tw, d), jnp.float32),
        pltpu.VMEM((tw, d), jnp.float32),
    ]

    @functools.partial(
        pl.kernel,
        out_type=jax.ShapeDtypeStruct((nw, d), jnp.float32),
        mesh=mesh,
        scratch_types=slot_types + slot_types + tail_types + [
            pltpu.VMEM((d,), jnp.float32),
        ],
    )
    def k(a_hbm, b_hbm, src_hbm, dst_hbm, cross_hbm,
          di0, si0, ra0, rb0, sd0, ss0, sa0, sb0,
          di1, si1, ra1, rb1, sd1, ss1, sa1, sb1,
          dit, sit, rat, rbt, st_c):
        cid = lax.axis_index("c")
        sid = lax.axis_index("s")
        wid = sid * nc + cid
        base = wid * epw
        slot0 = (di0, si0, ra0, rb0, sd0, ss0, sa0, sb0)
        slot1 = (di1, si1, ra1, rb1, sd1, ss1, sa1, sb1)
        zero = jnp.zeros((L,), jnp.float32)
        init = (zero,) * dsub

        def fire_idx(c, sl):
            di, si, _, _, sd, ss, _, _ = sl
            off = base + c * CH
            pltpu.async_copy(dst_hbm.at[pl.ds(off, CH)], di, sd)
            pltpu.async_copy(src_hbm.at[pl.ds(off, CH)], si, ss)

        def wait_idx(sl):
            di, si, _, _, sd, ss, _, _ = sl
            pltpu.make_async_copy(dst_hbm.at[pl.ds(0, CH)], di, sd).wait()
            pltpu.make_async_copy(src_hbm.at[pl.ds(0, CH)], si, ss).wait()

        def fire_gather(sl):
            di, si, ra, rb, _, _, sa, sb = sl
            pltpu.async_copy(a_hbm.at[di], ra, sa)
            pltpu.async_copy(b_hbm.at[si], rb, sb)

        def wait_gather(sl):
            di, si, ra, rb, _, _, sa, sb = sl
            pltpu.make_async_copy(a_hbm.at[di], ra, sa).wait()
            pltpu.make_async_copy(b_hbm.at[si], rb, sb).wait()

        def accum(ra, rb, nrows, carry):
            def row(r, cy):
                c_list = list(cy)
                for kk in range(dsub):
                    a = ra[r, pl.ds(kk * L, L)]
                    bb = rb[r, pl.ds(kk * L, L)]
                    c_list[kk] = c_list[kk] + a * bb
                return tuple(c_list)

            return lax.fori_loop(0, nrows, row, carry)

        def stage(c, sl, other, carry):
            wait_gather(sl)

            @pl.when(c + 1 < nfull)
            def _():
                wait_idx(other)
                fire_gather(other)

            carry = accum(sl[2], sl[3], CH, carry)

            @pl.when(c + 2 < nfull)
            def _():
                fire_idx(c + 2, sl)

            return carry

        if nfull >= 2:
            fire_idx(0, slot0)
            fire_idx(1, slot1)
            wait_idx(slot0)
            fire_gather(slot0)

            def pair(p, carry):
                carry = stage(2 * p, slot0, slot1, carry)
                return stage(2 * p + 1, slot1, slot0, carry)

            carry = lax.fori_loop(0, npair, pair, init)
            if odd:
                c = nfull - 1
                wait_gather(slot0 if c % 2 == 0 else slot1)
                sl = slot0 if c % 2 == 0 else slot1
                carry = accum(sl[2], sl[3], CH, carry)
        else:
            carry = init
            for c in range(nfull):
                off = base + c * CH
                pltpu.sync_copy(dst_hbm.at[pl.ds(off, CH)], di0)
                pltpu.sync_copy(src_hbm.at[pl.ds(off, CH)], si0)
                pltpu.async_copy(a_hbm.at[di0], ra0, sa0).wait()
                pltpu.async_copy(b_hbm.at[si0], rb0, sb0).wait()
                carry = accum(ra0, rb0, CH, carry)

        if tail:
            off = base + nfull * CH
            pltpu.sync_copy(dst_hbm.at[pl.ds(off, tail)], dit)
            pltpu.sync_copy(src_hbm.at[pl.ds(off, tail)], sit)
            pltpu.sync_copy(a_hbm.at[dit], rat)
            pltpu.sync_copy(b_hbm.at[sit], rbt)
            carry = accum(rat, rbt, tail, carry)

        for kk in range(dsub):
            st_c[pl.ds(kk * L, L)] = carry[kk]
        pltpu.sync_copy(st_c, cross_hbm.at[wid])

    return k(a_tab, b_tab, src, dst)


def _pass2(a_tab, b_tab, src, dst, st, zeros_hbm):
    n, d = a_tab.shape
    e = src.shape[0]
    dsub = d // L
    info = plsc.get_sparse_core_info()
    nc, ns = info.num_cores, info.num_subcores
    nw = nc * ns
    epw = e // nw
    ch = CH2
    nfull = epw // ch
    tail = epw % ch
    npair = nfull // 2
    odd = nfull % 2
    mesh = plsc.VectorSubcoreMesh(core_axis_name="c", subcore_axis_name="s")

    slot_types = [
        pltpu.VMEM((ch,), jnp.int32),
        pltpu.VMEM((ch,), jnp.int32),
        pltpu.VMEM((ch, d), jnp.float32),
        pltpu.VMEM((ch, d), jnp.float32),
        pltpu.SemaphoreType.DMA,
        pltpu.SemaphoreType.DMA,
        pltpu.SemaphoreType.DMA,
        pltpu.SemaphoreType.DMA,
        pltpu.VMEM((ch,), jnp.int32),
        pltpu.SemaphoreType.DMA,
    ]
    tw = tail if tail else 8
    tail_types = [
        pltpu.VMEM((tw,), jnp.int32),
        pltpu.VMEM((tw,), jnp.int32),
        pltpu.VMEM((tw, d), jnp.float32),
        pltpu.VMEM((tw, d), jnp.float32),
    ]

    @functools.partial(
        pl.kernel,
        out_type=jax.ShapeDtypeStruct((nc, n, d), jnp.float32),
        mesh=mesh,
        scratch_types=slot_types + slot_types + tail_types + [
            pltpu.VMEM((2, d), jnp.float32),
            pltpu.VMEM_SHARED((n, d), jnp.float32),
        ],
    )
    def k(a_hbm, b_hbm, src_hbm, dst_hbm, st_hbm, z_hbm, part_hbm,
          di0, si0, ra0, rb0, sd0, ss0, sa0, sb0, dx0, sy0,
          di1, si1, ra1, rb1, sd1, ss1, sa1, sb1, dx1, sy1,
          dit, sit, rat, rbt, stbuf, acc):
        cid = lax.axis_index("c")
        sid = lax.axis_index("s")
        wid = sid * nc + cid
        base = wid * epw
        slot0 = (di0, si0, ra0, rb0, sd0, ss0, sa0, sb0, dx0, sy0)
        slot1 = (di1, si1, ra1, rb1, sd1, ss1, sa1, sb1, dx1, sy1)

        @pl.when(sid == 0)
        def _():
            pltpu.sync_copy(z_hbm, acc)

        pltpu.sync_copy(st_hbm, stbuf)
        svals = [stbuf[0, pl.ds(kk * L, L)] for kk in range(dsub)]
        tvals = [stbuf[1, pl.ds(kk * L, L)] for kk in range(dsub)]
        plsc.subcore_barrier()

        def fire_idx(c, sl):
            di, si, sd, ss = sl[0], sl[1], sl[4], sl[5]
            off = base + c * ch
            pltpu.async_copy(dst_hbm.at[pl.ds(off, ch)], di, sd)
            pltpu.async_copy(src_hbm.at[pl.ds(off, ch)], si, ss)

        def wait_idx(sl):
            di, si, sd, ss = sl[0], sl[1], sl[4], sl[5]
            pltpu.make_async_copy(dst_hbm.at[pl.ds(0, ch)], di, sd).wait()
            pltpu.make_async_copy(src_hbm.at[pl.ds(0, ch)], si, ss).wait()

        def fire_gather(sl):
            di, si, ra, rb, sa, sb = sl[0], sl[1], sl[2], sl[3], sl[6], sl[7]
            pltpu.async_copy(a_hbm.at[di], ra, sa)
            pltpu.async_copy(b_hbm.at[si], rb, sb)

        def wait_gather(sl):
            di, si, ra, rb, sa, sb = sl[0], sl[1], sl[2], sl[3], sl[6], sl[7]
            pltpu.make_async_copy(a_hbm.at[di], ra, sa).wait()
            pltpu.make_async_copy(b_hbm.at[si], rb, sb).wait()

        def fire_scatter(sl):
            for kk in range(ch // L):
                sl[8][pl.ds(kk * L, L)] = sl[0][pl.ds(kk * L, L)]
            pltpu.async_copy(sl[2], acc.at[sl[8]], sl[9], add=True)

        def wait_scatter(sl):
            pltpu.make_async_copy(sl[2], acc.at[sl[8]], sl[9]).wait()

        def emit(ra, rb, nrows):
            def row(r, _r):
                for kk in range(dsub):
                    a = ra[r, pl.ds(kk * L, L)]
                    bb = rb[r, pl.ds(kk * L, L)]
                    y = (a + bb) * svals[kk] + tvals[kk]
                    y = jnp.maximum(y, 0.3 * y)
                    ra[r, pl.ds(kk * L, L)] = y
                return 0

            lax.fori_loop(0, nrows, row, 0)

        def stage(c, sl, other):
            wait_gather(sl)

            @pl.when(c + 1 < nfull)
            def _():
                wait_idx(other)

                @pl.when(c >= 1)
                def _():
                    wait_scatter(other)

                fire_gather(other)

            emit(sl[2], sl[3], ch)
            fire_scatter(sl)

            @pl.when(c + 2 < nfull)
            def _():
                fire_idx(c + 2, sl)

        if nfull >= 2:
            fire_idx(0, slot0)
            fire_idx(1, slot1)
            wait_idx(slot0)
            fire_gather(slot0)

            def pair(p, _):
                stage(2 * p, slot0, slot1)
                stage(2 * p + 1, slot1, slot0)
                return 0

            lax.fori_loop(0, npair, pair, 0)
            if odd:
                wait_scatter(slot1)
                c = nfull - 1
                sl = slot0 if c % 2 == 0 else slot1
                wait_gather(sl)
                emit(sl[2], sl[3], ch)
                pltpu.sync_copy(sl[2], acc.at[sl[0]], add=True)
            else:
                wait_scatter(slot0)
                wait_scatter(slot1)
        else:
            for c in range(nfull):
                off = base + c * ch
                pltpu.sync_copy(dst_hbm.at[pl.ds(off, ch)], di0)
                pltpu.sync_copy(src_hbm.at[pl.ds(off, ch)], si0)
                pltpu.async_copy(a_hbm.at[di0], ra0, sa0).wait()
                pltpu.async_copy(b_hbm.at[si0], rb0, sb0).wait()
                emit(ra0, rb0, ch)
                pltpu.sync_copy(ra0, acc.at[di0], add=True)

        if tail:
            off = base + nfull * ch
            pltpu.sync_copy(dst_hbm.at[pl.ds(off, tail)], dit)
            pltpu.sync_copy(src_hbm.at[pl.ds(off, tail)], sit)
            pltpu.sync_copy(a_hbm.at[dit], rat)
            pltpu.sync_copy(b_hbm.at[sit], rbt)
            emit(rat, rbt, tail)
            pltpu.sync_copy(rat, acc.at[dit], add=True)

        plsc.subcore_barrier()

        @pl.when(sid == 0)
        def _():
            pltpu.sync_copy(acc, part_hbm.at[cid])

    return k(a_tab, b_tab, src, dst, st, zeros_hbm)


def kernel(feature, edge_index, W, b, gamma, beta):
    n, d = feature.shape
    e = edge_index.shape[1]
    src = edge_index[0]
    dst = edge_index[1]
    w1 = W[:, :d]
    w2 = W[:, d:]
    wdt = (w1 - w2).T
    w2t = w2.T
    b2d = b.reshape(1, d)
    a_tab, b_tab = _prep(feature, wdt, w2t, b2d)
    cross = _pass1(a_tab, b_tab, src, dst)
    cnt2 = _count(dst, n)
    cnts2 = _count(src, n)
    st = _stats(a_tab, b_tab, cnt2, cnts2, cross,
                gamma.reshape(1, d), beta.reshape(1, d), float(e))
    zeros_hbm = jnp.zeros((n, d), jnp.float32)
    part = _pass2(a_tab, b_tab, src, dst, st, zeros_hbm)
    return _finalize(part[0], part[1], cnt2)


# speedup vs baseline: 1.3573x; 1.3573x over previous
import functools

import jax
import jax.numpy as jnp
from jax import lax
from jax.experimental import pallas as pl
from jax.experimental.pallas import tpu as pltpu
from jax.experimental.pallas import tpu_sc as plsc

L = 16
CH = 128
CH2 = 80
BR = 1000


def _prep_body(f_ref, wdt_ref, w2t_ref, b_ref, a_ref, bt_ref):
    f = f_ref[...]
    a_ref[...] = (
        jnp.dot(f, wdt_ref[...], preferred_element_type=jnp.float32) + b_ref[...]
    )
    bt_ref[...] = jnp.dot(f, w2t_ref[...], preferred_element_type=jnp.float32)


def _prep(feature, wdt, w2t, b2d):
    n, d = feature.shape
    grid = (n // BR,)
    return pl.pallas_call(
        _prep_body,
        grid=grid,
        in_specs=[
            pl.BlockSpec((BR, d), lambda i: (i, 0)),
            pl.BlockSpec((d, d), lambda i: (0, 0)),
            pl.BlockSpec((d, d), lambda i: (0, 0)),
            pl.BlockSpec((1, d), lambda i: (0, 0)),
        ],
        out_specs=[
            pl.BlockSpec((BR, d), lambda i: (i, 0)),
            pl.BlockSpec((BR, d), lambda i: (i, 0)),
        ],
        out_shape=[jax.ShapeDtypeStruct((n, d), jnp.float32)] * 2,
    )(feature, wdt, w2t, b2d)


def _stats(sums, sq, gamma2d, beta2d, num_edges):
    nw, d = sums.shape

    def body(s_ref, q_ref, g_ref, be_ref, st_ref):
        mean = jnp.sum(s_ref[...], axis=0, keepdims=True) / num_edges
        ex2 = jnp.sum(q_ref[...], axis=0, keepdims=True) / num_edges
        var = ex2 - mean * mean
        s = g_ref[...] * lax.rsqrt(var + 1e-5)
        t = be_ref[...] - mean * s
        st_ref[...] = jnp.concatenate([s, t], axis=0)

    return pl.pallas_call(
        body,
        out_shape=jax.ShapeDtypeStruct((2, d), jnp.float32),
    )(sums, sq, gamma2d, beta2d)


def _count(dst, n):
    e = dst.shape[0]
    eb = 2000
    grid = (e // eb,)
    dst3 = dst.reshape(e // eb, 1, eb)

    def body(d_ref, o_ref):
        i = pl.program_id(0)
        idx = d_ref[0, 0, :]
        hi = (idx >> 7).astype(jnp.int32)
        lo = (idx & 127).astype(jnp.int32)
        cols = lax.broadcasted_iota(jnp.int32, (eb, 128), 1)
        oh_hi = (hi[:, None] == cols).astype(jnp.float32)
        oh_lo = (lo[:, None] == cols).astype(jnp.float32)
        blk = lax.dot_general(
            oh_hi, oh_lo, (((0,), (0,)), ((), ())),
            preferred_element_type=jnp.float32,
        )

        @pl.when(i == 0)
        def _():
            o_ref[...] = jnp.zeros_like(o_ref)

        o_ref[...] += blk

    cnt2d = pl.pallas_call(
        body,
        grid=grid,
        in_specs=[pl.BlockSpec((1, 1, eb), lambda i: (i, 0, 0))],
        out_specs=pl.BlockSpec((128, 128), lambda i: (0, 0)),
        out_shape=jax.ShapeDtypeStruct((128, 128), jnp.float32),
    )(dst3)
    return cnt2d.reshape(-1)[:n].reshape(n, 1)


def _finalize(p0, p1, cnt2):
    n, d = p0.shape
    grid = (n // BR,)

    def body(p0_ref, p1_ref, c_ref, o_ref):
        p = p0_ref[...] + p1_ref[...]
        cnt = jnp.maximum(c_ref[...], 1.0)
        o_ref[...] = p * (1.0 / cnt)

    return pl.pallas_call(
        body,
        grid=grid,
        in_specs=[
            pl.BlockSpec((BR, d), lambda i: (i, 0)),
            pl.BlockSpec((BR, d), lambda i: (i, 0)),
            pl.BlockSpec((BR, 1), lambda i: (i, 0)),
        ],
        out_specs=pl.BlockSpec((BR, d), lambda i: (i, 0)),
        out_shape=jax.ShapeDtypeStruct((n, d), jnp.float32),
    )(p0, p1, cnt2)


def _pass1(a_tab, b_tab, src, dst):
    n, d = a_tab.shape
    e = src.shape[0]
    dsub = d // L
    info = plsc.get_sparse_core_info()
    nc, ns = info.num_cores, info.num_subcores
    nw = nc * ns
    epw = e // nw
    nfull = epw // CH
    tail = epw % CH
    npair = nfull // 2
    odd = nfull % 2
    mesh = plsc.VectorSubcoreMesh(core_axis_name="c", subcore_axis_name="s")

    slot_types = [
        pltpu.VMEM((CH,), jnp.int32),
        pltpu.VMEM((CH,), jnp.int32),
        pltpu.VMEM((CH, d), jnp.float32),
        pltpu.VMEM((CH, d), jnp.float32),
        pltpu.SemaphoreType.DMA,
        pltpu.SemaphoreType.DMA,
        pltpu.SemaphoreType.DMA,
        pltpu.SemaphoreType.DMA,
    ]
    tw = tail if tail else 8
    tail_types = [
        pltpu.VMEM((tw,), jnp.int32),
        pltpu.VMEM((tw,), jnp.int32),
        pltpu.VMEM((tw, d), jnp.float32),
        pltpu.VMEM((tw, d), jnp.float32),
    ]

    @functools.partial(
        pl.kernel,
        out_type=[
            jax.ShapeDtypeStruct((nw, d), jnp.float32),
            jax.ShapeDtypeStruct((nw, d), jnp.float32),
        ],
        mesh=mesh,
        scratch_types=slot_types + slot_types + tail_types + [
            pltpu.VMEM((d,), jnp.float32),
            pltpu.VMEM((d,), jnp.float32),
        ],
    )
    def k(a_hbm, b_hbm, src_hbm, dst_hbm, sums_hbm, sq_hbm,
          di0, si0, ra0, rb0, sd0, ss0, sa0, sb0,
          di1, si1, ra1, rb1, sd1, ss1, sa1, sb1,
          dit, sit, rat, rbt, st_s, st_q):
        cid = lax.axis_index("c")
        sid = lax.axis_index("s")
        wid = sid * nc + cid
        base = wid * epw
        slot0 = (di0, si0, ra0, rb0, sd0, ss0, sa0, sb0)
        slot1 = (di1, si1, ra1, rb1, sd1, ss1, sa1, sb1)
        zero = jnp.zeros((L,), jnp.float32)
        init = (zero,) * (2 * dsub)

        def fire_idx(c, sl):
            di, si, _, _, sd, ss, _, _ = sl
            off = base + c * CH
            pltpu.async_copy(dst_hbm.at[pl.ds(off, CH)], di, sd)
            pltpu.async_copy(src_hbm.at[pl.ds(off, CH)], si, ss)

        def wait_idx(sl):
            di, si, _, _, sd, ss, _, _ = sl
            pltpu.make_async_copy(dst_hbm.at[pl.ds(0, CH)], di, sd).wait()
            pltpu.make_async_copy(src_hbm.at[pl.ds(0, CH)], si, ss).wait()

        def fire_gather(sl):
            di, si, ra, rb, _, _, sa, sb = sl
            pltpu.async_copy(a_hbm.at[di], ra, sa)
            pltpu.async_copy(b_hbm.at[si], rb, sb)

        def wait_gather(sl):
            di, si, ra, rb, _, _, sa, sb = sl
            pltpu.make_async_copy(a_hbm.at[di], ra, sa).wait()
            pltpu.make_async_copy(b_hbm.at[si], rb, sb).wait()

        def accum(ra, rb, nrows, carry):
            def row(r, cy):
                s_list = list(cy[:dsub])
                q_list = list(cy[dsub:])
                for kk in range(dsub):
                    a = ra[r, pl.ds(kk * L, L)]
                    bb = rb[r, pl.ds(kk * L, L)]
                    h = a + bb
                    s_list[kk] = s_list[kk] + h
                    q_list[kk] = q_list[kk] + h * h
                return tuple(s_list) + tuple(q_list)

            return lax.fori_loop(0, nrows, row, carry)

        def stage(c, sl, other, carry):
            wait_gather(sl)

            @pl.when(c + 1 < nfull)
            def _():
                wait_idx(other)
                fire_gather(other)

            carry = accum(sl[2], sl[3], CH, carry)

            @pl.when(c + 2 < nfull)
            def _():
                fire_idx(c + 2, sl)

            return carry

        if nfull >= 2:
            fire_idx(0, slot0)
            fire_idx(1, slot1)
            wait_idx(slot0)
            fire_gather(slot0)

            def pair(p, carry):
                carry = stage(2 * p, slot0, slot1, carry)
                return stage(2 * p + 1, slot1, slot0, carry)

            carry = lax.fori_loop(0, npair, pair, init)
            if odd:
                c = nfull - 1
                wait_gather(slot0 if c % 2 == 0 else slot1)
                sl = slot0 if c % 2 == 0 else slot1
                carry = accum(sl[2], sl[3], CH, carry)
        else:
            carry = init
            for c in range(nfull):
                off = base + c * CH
                pltpu.sync_copy(dst_hbm.at[pl.ds(off, CH)], di0)
                pltpu.sync_copy(src_hbm.at[pl.ds(off, CH)], si0)
                pltpu.async_copy(a_hbm.at[di0], ra0, sa0).wait()
                pltpu.async_copy(b_hbm.at[si0], rb0, sb0).wait()
                carry = accum(ra0, rb0, CH, carry)

        if tail:
            off = base + nfull * CH
            pltpu.sync_copy(dst_hbm.at[pl.ds(off, tail)], dit)
            pltpu.sync_copy(src_hbm.at[pl.ds(off, tail)], sit)
            pltpu.sync_copy(a_hbm.at[dit], rat)
            pltpu.sync_copy(b_hbm.at[sit], rbt)
            carry = accum(rat, rbt, tail, carry)

        for kk in range(dsub):
            st_s[pl.ds(kk * L, L)] = carry[kk]
            st_q[pl.ds(kk * L, L)] = carry[dsub + kk]
        pltpu.sync_copy(st_s, sums_hbm.at[wid])
        pltpu.sync_copy(st_q, sq_hbm.at[wid])

    return k(a_tab, b_tab, src, dst)


def _pass2(a_tab, b_tab, src, dst, st, zeros_hbm):
    n, d = a_tab.shape
    e = src.shape[0]
    dsub = d // L
    info = plsc.get_sparse_core_info()
    nc, ns = info.num_cores, info.num_subcores
    nw = nc * ns
    epw = e // nw
    ch = CH2
    nfull = epw // ch
    tail = epw % ch
    npair = nfull // 2
    odd = nfull % 2
    mesh = plsc.VectorSubcoreMesh(core_axis_name="c", subcore_axis_name="s")

    slot_types = [
        pltpu.VMEM((ch,), jnp.int32),
        pltpu.VMEM((ch,), jnp.int32),
        pltpu.VMEM((ch, d), jnp.float32),
        pltpu.VMEM((ch, d), jnp.float32),
        pltpu.SemaphoreType.DMA,
        pltpu.SemaphoreType.DMA,
        pltpu.SemaphoreType.DMA,
        pltpu.SemaphoreType.DMA,
        pltpu.VMEM((ch,), jnp.int32),
        pltpu.SemaphoreType.DMA,
    ]
    tw = tail if tail else 8
    tail_types = [
        pltpu.VMEM((tw,), jnp.int32),
        pltpu.VMEM((tw,), jnp.int32),
        pltpu.VMEM((tw, d), jnp.float32),
        pltpu.VMEM((tw, d), jnp.float32),
    ]

    @functools.partial(
        pl.kernel,
        out_type=jax.ShapeDtypeStruct((nc, n, d), jnp.float32),
        mesh=mesh,
        scratch_types=slot_types + slot_types + tail_types + [
            pltpu.VMEM((2, d), jnp.float32),
            pltpu.VMEM_SHARED((n, d), jnp.float32),
        ],
    )
    def k(a_hbm, b_hbm, src_hbm, dst_hbm, st_hbm, z_hbm, part_hbm,
          di0, si0, ra0, rb0, sd0, ss0, sa0, sb0, dx0, sy0,
          di1, si1, ra1, rb1, sd1, ss1, sa1, sb1, dx1, sy1,
          dit, sit, rat, rbt, stbuf, acc):
        cid = lax.axis_index("c")
        sid = lax.axis_index("s")
        wid = sid * nc + cid
        base = wid * epw
        slot0 = (di0, si0, ra0, rb0, sd0, ss0, sa0, sb0, dx0, sy0)
        slot1 = (di1, si1, ra1, rb1, sd1, ss1, sa1, sb1, dx1, sy1)

        @pl.when(sid == 0)
        def _():
            pltpu.sync_copy(z_hbm, acc)

        pltpu.sync_copy(st_hbm, stbuf)
        svals = [stbuf[0, pl.ds(kk * L, L)] for kk in range(dsub)]
        tvals = [stbuf[1, pl.ds(kk * L, L)] for kk in range(dsub)]
        plsc.subcore_barrier()

        def fire_idx(c, sl):
            di, si, sd, ss = sl[0], sl[1], sl[4], sl[5]
            off = base + c * ch
            pltpu.async_copy(dst_hbm.at[pl.ds(off, ch)], di, sd)
            pltpu.async_copy(src_hbm.at[pl.ds(off, ch)], si, ss)

        def wait_idx(sl):
            di, si, sd, ss = sl[0], sl[1], sl[4], sl[5]
            pltpu.make_async_copy(dst_hbm.at[pl.ds(0, ch)], di, sd).wait()
            pltpu.make_async_copy(src_hbm.at[pl.ds(0, ch)], si, ss).wait()

        def fire_gather(sl):
            di, si, ra, rb, sa, sb = sl[0], sl[1], sl[2], sl[3], sl[6], sl[7]
            pltpu.async_copy(a_hbm.at[di], ra, sa)
            pltpu.async_copy(b_hbm.at[si], rb, sb)

        def wait_gather(sl):
            di, si, ra, rb, sa, sb = sl[0], sl[1], sl[2], sl[3], sl[6], sl[7]
            pltpu.make_async_copy(a_hbm.at[di], ra, sa).wait()
            pltpu.make_async_copy(b_hbm.at[si], rb, sb).wait()

        def fire_scatter(sl):
            for kk in range(ch // L):
                sl[8][pl.ds(kk * L, L)] = sl[0][pl.ds(kk * L, L)]
            pltpu.async_copy(sl[2], acc.at[sl[8]], sl[9], add=True)

        def wait_scatter(sl):
            pltpu.make_async_copy(sl[2], acc.at[sl[8]], sl[9]).wait()

        def emit(ra, rb, nrows):
            def row(r, _r):
                for kk in range(dsub):
                    a = ra[r, pl.ds(kk * L, L)]
                    bb = rb[r, pl.ds(kk * L, L)]
                    y = (a + bb) * svals[kk] + tvals[kk]
                    y = jnp.maximum(y, 0.3 * y)
                    ra[r, pl.ds(kk * L, L)] = y
                return 0

            lax.fori_loop(0, nrows, row, 0)

        def stage(c, sl, other):
            wait_gather(sl)

            @pl.when(c + 1 < nfull)
            def _():
                wait_idx(other)

                @pl.when(c >= 1)
                def _():
                    wait_scatter(other)

                fire_gather(other)

            emit(sl[2], sl[3], ch)
            fire_scatter(sl)

            @pl.when(c + 2 < nfull)
            def _():
                fire_idx(c + 2, sl)

        if nfull >= 2:
            fire_idx(0, slot0)
            fire_idx(1, slot1)
            wait_idx(slot0)
            fire_gather(slot0)

            def pair(p, _):
                stage(2 * p, slot0, slot1)
                stage(2 * p + 1, slot1, slot0)
                return 0

            lax.fori_loop(0, npair, pair, 0)
            if odd:
                wait_scatter(slot1)
                c = nfull - 1
                sl = slot0 if c % 2 == 0 else slot1
                wait_gather(sl)
                emit(sl[2], sl[3], ch)
                pltpu.sync_copy(sl[2], acc.at[sl[0]], add=True)
            else:
                wait_scatter(slot0)
                wait_scatter(slot1)
        else:
            for c in range(nfull):
                off = base + c * ch
                pltpu.sync_copy(dst_hbm.at[pl.ds(off, ch)], di0)
                pltpu.sync_copy(src_hbm.at[pl.ds(off, ch)], si0)
                pltpu.async_copy(a_hbm.at[di0], ra0, sa0).wait()
                pltpu.async_copy(b_hbm.at[si0], rb0, sb0).wait()
                emit(ra0, rb0, ch)
                pltpu.sync_copy(ra0, acc.at[di0], add=True)

        if tail:
            off = base + nfull * ch
            pltpu.sync_copy(dst_hbm.at[pl.ds(off, tail)], dit)
            pltpu.sync_copy(src_hbm.at[pl.ds(off, tail)], sit)
            pltpu.sync_copy(a_hbm.at[dit], rat)
            pltpu.sync_copy(b_hbm.at[sit], rbt)
            emit(rat, rbt, tail)
            pltpu.sync_copy(rat, acc.at[dit], add=True)

        plsc.subcore_barrier()

        @pl.when(sid == 0)
        def _():
            pltpu.sync_copy(acc, part_hbm.at[cid])

    return k(a_tab, b_tab, src, dst, st, zeros_hbm)


def kernel(feature, edge_index, W, b, gamma, beta):
    n, d = feature.shape
    e = edge_index.shape[1]
    src = edge_index[0]
    dst = edge_index[1]
    w1 = W[:, :d]
    w2 = W[:, d:]
    wdt = (w1 - w2).T
    w2t = w2.T
    b2d = b.reshape(1, d)
    a_tab, b_tab = _prep(feature, wdt, w2t, b2d)
    sums, sq = _pass1(a_tab, b_tab, src, dst)
    st = _stats(sums, sq, gamma.reshape(1, d), beta.reshape(1, d), float(e))
    zeros_hbm = jnp.zeros((n, d), jnp.float32)
    part = _pass2(a_tab, b_tab, src, dst, st, zeros_hbm)
    cnt2 = _count(dst, n)
    return _finalize(part[0], part[1], cnt2)
